# baseline (speedup 1.0000x reference)
import functools

import jax
import jax.numpy as jnp
from jax import lax
from jax.experimental import pallas as pl
from jax.experimental.pallas import tpu as pltpu
from jax.experimental.pallas import tpu_sc as plsc

_NUM_SPEAKERS = 100000
_DIM = 64
_BATCH = 16384


@functools.cache
def _make_gather(V, D, B):
    info = plsc.get_sparse_core_info()
    NC, NS = info.num_cores, info.num_subcores
    NW = NC * NS
    assert B % NW == 0
    b_per_w = B // NW
    mesh = plsc.VectorSubcoreMesh(core_axis_name="c", subcore_axis_name="s")

    @functools.partial(
        pl.kernel,
        mesh=mesh,
        out_type=jax.ShapeDtypeStruct((B, D), jnp.float32),
        scratch_types=[
            pltpu.VMEM((b_per_w,), jnp.int32),
            pltpu.SemaphoreType.DMA,
        ],
    )
    def gather_kernel(table_hbm, idx_hbm, out_hbm, idx_v, sem):
        wid = lax.axis_index("s") * NC + lax.axis_index("c")
        base = wid * b_per_w
        pltpu.sync_copy(idx_hbm.at[pl.ds(base, b_per_w)], idx_v)

        def fire(g, carry):
            start = pl.multiple_of(g * 16, 16)
            idxs = idx_v[pl.ds(start, 16)]
            for l in range(16):
                pltpu.async_copy(
                    table_hbm.at[pl.ds(idxs[l], 1), :],
                    out_hbm.at[pl.ds(base + start + l, 1), :],
                    sem,
                )
            return carry

        lax.fori_loop(0, b_per_w // 16, fire, 0)

        def drain(b, carry):
            pltpu.make_async_copy(
                table_hbm.at[pl.ds(0, 1), :],
                out_hbm.at[pl.ds(base + b, 1), :],
                sem,
            ).wait()
            return carry

        lax.fori_loop(0, b_per_w, drain, 0)

    return gather_kernel


@jax.jit
def kernel(spk_ids, table):
    gather = _make_gather(_NUM_SPEAKERS, _DIM, _BATCH)
    return gather(table, spk_ids.astype(jnp.int32))


# speedup vs baseline: 3.6515x; 3.6515x over previous
import functools

import jax
import jax.numpy as jnp
from jax import lax
from jax.experimental import pallas as pl
from jax.experimental.pallas import tpu as pltpu
from jax.experimental.pallas import tpu_sc as plsc

_NUM_SPEAKERS = 100000
_DIM = 64
_BATCH = 16384
_DPAD = 128


@functools.cache
def _make_gather(V, D, B):
    info = plsc.get_sparse_core_info()
    NC, NS = info.num_cores, info.num_subcores
    NW = NC * NS
    assert B % NW == 0
    b_per_w = B // NW
    mesh = plsc.VectorSubcoreMesh(core_axis_name="c", subcore_axis_name="s")

    @functools.partial(
        pl.kernel,
        mesh=mesh,
        out_type=jax.ShapeDtypeStruct((B, _DPAD), jnp.float32),
        scratch_types=[
            pltpu.VMEM((b_per_w,), jnp.int32),
            pltpu.VMEM((2, b_per_w // 2, _DPAD), jnp.float32),
            pltpu.SemaphoreType.DMA,
            pltpu.SemaphoreType.DMA,
        ],
        compiler_params=pltpu.CompilerParams(skip_device_barrier=True),
    )
    def gather_kernel(table_hbm, idx_hbm, out_hbm, idx_v, rows_v, gsem, wsem):
        wid = lax.axis_index("s") * NC + lax.axis_index("c")
        base = wid * b_per_w
        half = b_per_w // 2
        pltpu.sync_copy(idx_hbm.at[pl.ds(base, b_per_w)], idx_v)
        g0 = pltpu.async_copy(
            table_hbm.at[idx_v.at[pl.ds(0, half)]], rows_v.at[0], gsem
        )
        g1 = pltpu.async_copy(
            table_hbm.at[idx_v.at[pl.ds(half, half)]], rows_v.at[1], gsem
        )
        g0.wait()
        w0 = pltpu.async_copy(rows_v.at[0], out_hbm.at[pl.ds(base, half)], wsem)
        g1.wait()
        w1 = pltpu.async_copy(
            rows_v.at[1], out_hbm.at[pl.ds(base + half, half)], wsem
        )
        w0.wait()
        w1.wait()

    return gather_kernel


@jax.jit
def kernel(spk_ids, table):
    gather = _make_gather(_NUM_SPEAKERS, _DIM, _BATCH)
    table_pad = jnp.pad(table, ((0, 0), (0, _DPAD - _DIM)))
    out_pad = gather(table_pad, spk_ids.astype(jnp.int32))
    return out_pad[:, :_DIM]


# speedup vs baseline: 3.7916x; 1.0384x over previous
import functools

import jax
import jax.numpy as jnp
from jax import lax
from jax.experimental import pallas as pl
from jax.experimental.pallas import tpu as pltpu
from jax.experimental.pallas import tpu_sc as plsc

_NUM_SPEAKERS = 100000
_DIM = 64
_BATCH = 16384
_DPAD = 128


@functools.cache
def _make_gather(V, D, B):
    info = plsc.get_sparse_core_info()
    NC, NS = info.num_cores, info.num_subcores
    NW = NC * NS
    assert B % NW == 0
    b_per_w = B // NW
    half = b_per_w // 2
    mesh = plsc.VectorSubcoreMesh(core_axis_name="c", subcore_axis_name="s")

    @functools.partial(
        pl.kernel,
        mesh=mesh,
        out_type=jax.ShapeDtypeStruct((B, _DPAD), jnp.float32),
        scratch_types=[
            pltpu.VMEM((b_per_w,), jnp.int32),
            pltpu.VMEM((2, half, D), jnp.float32),
            pltpu.SemaphoreType.DMA,
            pltpu.SemaphoreType.DMA,
        ],
        compiler_params=pltpu.CompilerParams(
            skip_device_barrier=True, use_tc_tiling_on_sc=False
        ),
    )
    def gather_kernel(table_hbm, idx_hbm, out_hbm, idx_v, rows_v, gsem, wsem):
        wid = lax.axis_index("s") * NC + lax.axis_index("c")
        base = wid * b_per_w
        pltpu.sync_copy(idx_hbm.at[pl.ds(base, b_per_w)], idx_v)

        def scale(g, carry):
            start = pl.multiple_of(g * 16, 16)
            idx_v[pl.ds(start, 16)] = idx_v[pl.ds(start, 16)] << 1
            return carry

        lax.fori_loop(0, b_per_w // 16, scale, 0)

        g0 = pltpu.async_copy(
            table_hbm.at[idx_v.at[pl.ds(0, half)]], rows_v.at[0], gsem
        )
        g1 = pltpu.async_copy(
            table_hbm.at[idx_v.at[pl.ds(half, half)]], rows_v.at[1], gsem
        )
        g0.wait()
        w0 = pltpu.async_copy(
            rows_v.at[0], out_hbm.at[pl.ds(base, half), pl.ds(0, D)], wsem
        )
        g1.wait()
        w1 = pltpu.async_copy(
            rows_v.at[1], out_hbm.at[pl.ds(base + half, half), pl.ds(0, D)], wsem
        )
        w0.wait()
        w1.wait()

    return gather_kernel


@jax.jit
def kernel(spk_ids, table):
    gather = _make_gather(_NUM_SPEAKERS, _DIM, _BATCH)
    table_pad = jnp.pad(table, ((0, 0), (0, _DPAD - _DIM)))
    table_half = table_pad.reshape(2 * _NUM_SPEAKERS, _DIM)
    out_pad = gather(table_half, spk_ids.astype(jnp.int32))
    return out_pad[:, :_DIM]
